# initial kernel scaffold (unmeasured)
import jax
import jax.numpy as jnp
from jax import lax
from jax.experimental import pallas as pl
from jax.experimental.pallas import tpu as pltpu


def kernel(
    x,
):
    def body(*refs):
        pass

    out_shape = jax.ShapeDtypeStruct(..., jnp.float32)
    return pl.pallas_call(body, out_shape=out_shape)(...)


# baseline (device time: 123001 ns/iter reference)
import jax
import jax.numpy as jnp
from jax import lax
from jax.experimental import pallas as pl
from jax.experimental.pallas import tpu as pltpu

N_Y = 2


def kernel(x):
    xb = x.astype(jnp.bfloat16)
    m, n = xb.shape
    half_n = n // N_Y

    def body(x_ref, out_ref, send_sem, recv_sem):
        my_x = lax.axis_index("x")
        my_y = lax.axis_index("y")
        peer = (my_x, 1 - my_y)

        barrier_sem = pltpu.get_barrier_semaphore()
        pl.semaphore_signal(
            barrier_sem, inc=1,
            device_id=peer, device_id_type=pl.DeviceIdType.MESH,
        )
        pl.semaphore_wait(barrier_sem, 1)

        out_ref[pl.ds(my_y * m, m), :] = x_ref[:, pl.ds(my_y * half_n, half_n)]

        rdma = pltpu.make_async_remote_copy(
            src_ref=x_ref.at[:, pl.ds((1 - my_y) * half_n, half_n)],
            dst_ref=out_ref.at[pl.ds(my_y * m, m), :],
            send_sem=send_sem,
            recv_sem=recv_sem,
            device_id=peer,
            device_id_type=pl.DeviceIdType.MESH,
        )
        rdma.start()
        rdma.wait()

    return pl.pallas_call(
        body,
        out_shape=jax.ShapeDtypeStruct((N_Y * m, half_n), jnp.bfloat16),
        in_specs=[pl.BlockSpec(memory_space=pltpu.VMEM)],
        out_specs=pl.BlockSpec(memory_space=pltpu.VMEM),
        scratch_shapes=[
            pltpu.SemaphoreType.DMA,
            pltpu.SemaphoreType.DMA,
        ],
        compiler_params=pltpu.CompilerParams(collective_id=0),
    )(xb)


# device time: 86688 ns/iter; 1.4189x vs baseline; 1.4189x over previous
import jax
import jax.numpy as jnp
from jax import lax
from jax.experimental import pallas as pl
from jax.experimental.pallas import tpu as pltpu

N_Y = 2
N_CHUNKS = 8


def kernel(x):
    m, n = x.shape
    half_n = n // 2
    half_m = m // 2
    rows_c = half_m // N_CHUNKS

    def body(x_ref, out_ref, send_buf, ysend, yrecv, xsend, xrecv):
        my_x = lax.axis_index("x")
        my_y = lax.axis_index("y")
        y_nbr = (my_x, 1 - my_y)
        x_nbr = (1 - my_x, my_y)

        barrier_sem = pltpu.get_barrier_semaphore()
        for nbr in (y_nbr, x_nbr):
            pl.semaphore_signal(
                barrier_sem, inc=1,
                device_id=nbr, device_id_type=pl.DeviceIdType.MESH,
            )
        pl.semaphore_wait(barrier_sem, 2)

        send_buf[...] = x_ref[
            pl.ds(my_x * half_m, half_m), pl.ds((1 - my_y) * half_n, half_n)
        ].astype(jnp.bfloat16)

        y_rdmas = []
        for c in range(N_CHUNKS):
            rd = pltpu.make_async_remote_copy(
                src_ref=send_buf.at[pl.ds(c * rows_c, rows_c), :],
                dst_ref=out_ref.at[
                    pl.ds(my_y * m + my_x * half_m + c * rows_c, rows_c), :
                ],
                send_sem=ysend.at[c],
                recv_sem=yrecv.at[c],
                device_id=y_nbr,
                device_id_type=pl.DeviceIdType.MESH,
            )
            rd.start()
            y_rdmas.append(rd)

        out_ref[pl.ds(my_y * m, m), :] = x_ref[
            :, pl.ds(my_y * half_n, half_n)
        ].astype(jnp.bfloat16)

        x_rdmas = []
        for c in range(N_CHUNKS):
            row0 = (1 - my_y) * m + my_x * half_m + c * rows_c
            recv_view = pltpu.make_async_remote_copy(
                src_ref=send_buf.at[pl.ds(c * rows_c, rows_c), :],
                dst_ref=out_ref.at[pl.ds(row0, rows_c), :],
                send_sem=ysend.at[c],
                recv_sem=yrecv.at[c],
                device_id=y_nbr,
                device_id_type=pl.DeviceIdType.MESH,
            )
            recv_view.wait_recv()
            fwd = pltpu.make_async_remote_copy(
                src_ref=out_ref.at[pl.ds(row0, rows_c), :],
                dst_ref=out_ref.at[pl.ds(row0, rows_c), :],
                send_sem=xsend.at[c],
                recv_sem=xrecv.at[c],
                device_id=x_nbr,
                device_id_type=pl.DeviceIdType.MESH,
            )
            fwd.start()
            x_rdmas.append(fwd)

        for c in range(N_CHUNKS):
            x_rdmas[c].wait_recv()
        for c in range(N_CHUNKS):
            y_rdmas[c].wait_send()
            x_rdmas[c].wait_send()

    return pl.pallas_call(
        body,
        out_shape=jax.ShapeDtypeStruct((N_Y * m, half_n), jnp.bfloat16),
        in_specs=[pl.BlockSpec(memory_space=pltpu.VMEM)],
        out_specs=pl.BlockSpec(memory_space=pltpu.VMEM),
        scratch_shapes=[
            pltpu.VMEM((half_m, half_n), jnp.bfloat16),
            pltpu.SemaphoreType.DMA((N_CHUNKS,)),
            pltpu.SemaphoreType.DMA((N_CHUNKS,)),
            pltpu.SemaphoreType.DMA((N_CHUNKS,)),
            pltpu.SemaphoreType.DMA((N_CHUNKS,)),
        ],
        compiler_params=pltpu.CompilerParams(
            collective_id=0,
            vmem_limit_bytes=96 * 1024 * 1024,
        ),
    )(x)


# device time: 74652 ns/iter; 1.6477x vs baseline; 1.1612x over previous
import jax
import jax.numpy as jnp
from jax import lax
from jax.experimental import pallas as pl
from jax.experimental.pallas import tpu as pltpu

N_Y = 2
N_CHUNKS = 8


def kernel(x):
    m, n = x.shape
    half_n = n // 2
    half_m = m // 2
    rows_c = half_m // N_CHUNKS

    def body(
        x_hbm, out_hbm,
        peer_f32, own_f32, own_bf16, send_buf,
        cp_sems, ysend, yrecv, xsend, xrecv,
    ):
        my_x = lax.axis_index("x")
        my_y = lax.axis_index("y")
        y_nbr = (my_x, 1 - my_y)
        x_nbr = (1 - my_x, my_y)

        cp_peer = pltpu.make_async_copy(
            x_hbm.at[pl.ds(my_x * half_m, half_m),
                     pl.ds((1 - my_y) * half_n, half_n)],
            peer_f32,
            cp_sems.at[0],
        )
        cp_peer.start()
        cp_own = pltpu.make_async_copy(
            x_hbm.at[:, pl.ds(my_y * half_n, half_n)],
            own_f32,
            cp_sems.at[1],
        )
        cp_own.start()

        barrier_sem = pltpu.get_barrier_semaphore()
        for nbr in (y_nbr, x_nbr):
            pl.semaphore_signal(
                barrier_sem, inc=1,
                device_id=nbr, device_id_type=pl.DeviceIdType.MESH,
            )
        pl.semaphore_wait(barrier_sem, 2)

        cp_peer.wait()
        send_buf[...] = peer_f32[...].astype(jnp.bfloat16)

        y_rdmas = []
        for c in range(N_CHUNKS):
            rd = pltpu.make_async_remote_copy(
                src_ref=send_buf.at[pl.ds(c * rows_c, rows_c), :],
                dst_ref=out_hbm.at[
                    pl.ds(my_y * m + my_x * half_m + c * rows_c, rows_c), :
                ],
                send_sem=ysend.at[c],
                recv_sem=yrecv.at[c],
                device_id=y_nbr,
                device_id_type=pl.DeviceIdType.MESH,
            )
            rd.start()
            y_rdmas.append(rd)

        cp_own.wait()
        own_bf16[...] = own_f32[...].astype(jnp.bfloat16)
        cp_out = pltpu.make_async_copy(
            own_bf16,
            out_hbm.at[pl.ds(my_y * m, m), :],
            cp_sems.at[2],
        )
        cp_out.start()

        x_rdmas = []
        for c in range(N_CHUNKS):
            row0 = (1 - my_y) * m + my_x * half_m + c * rows_c
            recv_view = pltpu.make_async_remote_copy(
                src_ref=send_buf.at[pl.ds(c * rows_c, rows_c), :],
                dst_ref=out_hbm.at[pl.ds(row0, rows_c), :],
                send_sem=ysend.at[c],
                recv_sem=yrecv.at[c],
                device_id=y_nbr,
                device_id_type=pl.DeviceIdType.MESH,
            )
            recv_view.wait_recv()
            fwd = pltpu.make_async_remote_copy(
                src_ref=out_hbm.at[pl.ds(row0, rows_c), :],
                dst_ref=out_hbm.at[pl.ds(row0, rows_c), :],
                send_sem=xsend.at[c],
                recv_sem=xrecv.at[c],
                device_id=x_nbr,
                device_id_type=pl.DeviceIdType.MESH,
            )
            fwd.start()
            x_rdmas.append(fwd)

        for c in range(N_CHUNKS):
            x_rdmas[c].wait_recv()
        for c in range(N_CHUNKS):
            y_rdmas[c].wait_send()
            x_rdmas[c].wait_send()
        cp_out.wait()

    return pl.pallas_call(
        body,
        out_shape=jax.ShapeDtypeStruct((N_Y * m, half_n), jnp.bfloat16),
        in_specs=[pl.BlockSpec(memory_space=pl.ANY)],
        out_specs=pl.BlockSpec(memory_space=pl.ANY),
        scratch_shapes=[
            pltpu.VMEM((half_m, half_n), jnp.float32),
            pltpu.VMEM((m, half_n), jnp.float32),
            pltpu.VMEM((m, half_n), jnp.bfloat16),
            pltpu.VMEM((half_m, half_n), jnp.bfloat16),
            pltpu.SemaphoreType.DMA((3,)),
            pltpu.SemaphoreType.DMA((N_CHUNKS,)),
            pltpu.SemaphoreType.DMA((N_CHUNKS,)),
            pltpu.SemaphoreType.DMA((N_CHUNKS,)),
            pltpu.SemaphoreType.DMA((N_CHUNKS,)),
        ],
        compiler_params=pltpu.CompilerParams(
            collective_id=0,
            vmem_limit_bytes=96 * 1024 * 1024,
        ),
    )(x)


# device time: 73632 ns/iter; 1.6705x vs baseline; 1.0139x over previous
import jax
import jax.numpy as jnp
from jax import lax
from jax.experimental import pallas as pl
from jax.experimental.pallas import tpu as pltpu

N_Y = 2
N_CHUNKS = 8


def kernel(x):
    m, n = x.shape
    half_n = n // 2
    half_m = m // 2
    rows_c = half_m // N_CHUNKS

    def body(
        x_hbm, out_hbm,
        peer_f32, own_f32, own_bf16, send_buf,
        cp_sems, peer_sems, ysend, yrecv, xsend, xrecv,
    ):
        my_x = lax.axis_index("x")
        my_y = lax.axis_index("y")
        y_nbr = (my_x, 1 - my_y)
        x_nbr = (1 - my_x, my_y)

        cp_peers = []
        for c in range(N_CHUNKS):
            cp = pltpu.make_async_copy(
                x_hbm.at[pl.ds(my_x * half_m + c * rows_c, rows_c),
                         pl.ds((1 - my_y) * half_n, half_n)],
                peer_f32.at[pl.ds(c * rows_c, rows_c), :],
                peer_sems.at[c],
            )
            cp.start()
            cp_peers.append(cp)

        barrier_sem = pltpu.get_barrier_semaphore()
        for nbr in (y_nbr, x_nbr):
            pl.semaphore_signal(
                barrier_sem, inc=1,
                device_id=nbr, device_id_type=pl.DeviceIdType.MESH,
            )
        pl.semaphore_wait(barrier_sem, 2)

        y_rdmas = []
        for c in range(N_CHUNKS):
            cp_peers[c].wait()
            send_buf[pl.ds(c * rows_c, rows_c), :] = peer_f32[
                pl.ds(c * rows_c, rows_c), :
            ].astype(jnp.bfloat16)
            rd = pltpu.make_async_remote_copy(
                src_ref=send_buf.at[pl.ds(c * rows_c, rows_c), :],
                dst_ref=out_hbm.at[
                    pl.ds(my_y * m + my_x * half_m + c * rows_c, rows_c), :
                ],
                send_sem=ysend.at[c],
                recv_sem=yrecv.at[c],
                device_id=y_nbr,
                device_id_type=pl.DeviceIdType.MESH,
            )
            rd.start()
            y_rdmas.append(rd)

        cp_own = pltpu.make_async_copy(
            x_hbm.at[:, pl.ds(my_y * half_n, half_n)],
            own_f32,
            cp_sems.at[0],
        )
        cp_own.start()
        cp_own.wait()
        own_bf16[...] = own_f32[...].astype(jnp.bfloat16)
        cp_out = pltpu.make_async_copy(
            own_bf16,
            out_hbm.at[pl.ds(my_y * m, m), :],
            cp_sems.at[1],
        )
        cp_out.start()

        x_rdmas = []
        for c in range(N_CHUNKS):
            row0 = (1 - my_y) * m + my_x * half_m + c * rows_c
            recv_view = pltpu.make_async_remote_copy(
                src_ref=send_buf.at[pl.ds(c * rows_c, rows_c), :],
                dst_ref=out_hbm.at[pl.ds(row0, rows_c), :],
                send_sem=ysend.at[c],
                recv_sem=yrecv.at[c],
                device_id=y_nbr,
                device_id_type=pl.DeviceIdType.MESH,
            )
            recv_view.wait_recv()
            fwd = pltpu.make_async_remote_copy(
                src_ref=out_hbm.at[pl.ds(row0, rows_c), :],
                dst_ref=out_hbm.at[pl.ds(row0, rows_c), :],
                send_sem=xsend.at[c],
                recv_sem=xrecv.at[c],
                device_id=x_nbr,
                device_id_type=pl.DeviceIdType.MESH,
            )
            fwd.start()
            x_rdmas.append(fwd)

        for c in range(N_CHUNKS):
            x_rdmas[c].wait_recv()
        for c in range(N_CHUNKS):
            y_rdmas[c].wait_send()
            x_rdmas[c].wait_send()
        cp_out.wait()

    return pl.pallas_call(
        body,
        out_shape=jax.ShapeDtypeStruct((N_Y * m, half_n), jnp.bfloat16),
        in_specs=[pl.BlockSpec(memory_space=pl.ANY)],
        out_specs=pl.BlockSpec(memory_space=pl.ANY),
        scratch_shapes=[
            pltpu.VMEM((half_m, half_n), jnp.float32),
            pltpu.VMEM((m, half_n), jnp.float32),
            pltpu.VMEM((m, half_n), jnp.bfloat16),
            pltpu.VMEM((half_m, half_n), jnp.bfloat16),
            pltpu.SemaphoreType.DMA((2,)),
            pltpu.SemaphoreType.DMA((N_CHUNKS,)),
            pltpu.SemaphoreType.DMA((N_CHUNKS,)),
            pltpu.SemaphoreType.DMA((N_CHUNKS,)),
            pltpu.SemaphoreType.DMA((N_CHUNKS,)),
            pltpu.SemaphoreType.DMA((N_CHUNKS,)),
        ],
        compiler_params=pltpu.CompilerParams(
            collective_id=0,
            vmem_limit_bytes=96 * 1024 * 1024,
        ),
    )(x)


# device time: 69976 ns/iter; 1.7578x vs baseline; 1.0522x over previous
import jax
import jax.numpy as jnp
from jax import lax
from jax.experimental import pallas as pl
from jax.experimental.pallas import tpu as pltpu

N_Y = 2
N_CHUNKS = 16


def kernel(x):
    m, n = x.shape
    half_n = n // 2
    half_m = m // 2
    rows_c = half_m // N_CHUNKS

    def body(
        x_hbm, out_hbm,
        peer_f32, own_f32, own_bf16, send_buf,
        cp_sems, peer_sems, ysend, yrecv, xsend, xrecv,
    ):
        my_x = lax.axis_index("x")
        my_y = lax.axis_index("y")
        y_nbr = (my_x, 1 - my_y)
        x_nbr = (1 - my_x, my_y)

        cp_peers = []
        for c in range(N_CHUNKS):
            cp = pltpu.make_async_copy(
                x_hbm.at[pl.ds(my_x * half_m + c * rows_c, rows_c),
                         pl.ds((1 - my_y) * half_n, half_n)],
                peer_f32.at[pl.ds(c * rows_c, rows_c), :],
                peer_sems.at[c],
            )
            cp.start()
            cp_peers.append(cp)

        barrier_sem = pltpu.get_barrier_semaphore()
        for nbr in (y_nbr, x_nbr):
            pl.semaphore_signal(
                barrier_sem, inc=1,
                device_id=nbr, device_id_type=pl.DeviceIdType.MESH,
            )
        pl.semaphore_wait(barrier_sem, 2)

        y_rdmas = []
        for c in range(N_CHUNKS):
            cp_peers[c].wait()
            send_buf[pl.ds(c * rows_c, rows_c), :] = peer_f32[
                pl.ds(c * rows_c, rows_c), :
            ].astype(jnp.bfloat16)
            rd = pltpu.make_async_remote_copy(
                src_ref=send_buf.at[pl.ds(c * rows_c, rows_c), :],
                dst_ref=out_hbm.at[
                    pl.ds(my_y * m + my_x * half_m + c * rows_c, rows_c), :
                ],
                send_sem=ysend.at[c],
                recv_sem=yrecv.at[c],
                device_id=y_nbr,
                device_id_type=pl.DeviceIdType.MESH,
            )
            rd.start()
            y_rdmas.append(rd)

        cp_own = pltpu.make_async_copy(
            x_hbm.at[:, pl.ds(my_y * half_n, half_n)],
            own_f32,
            cp_sems.at[0],
        )
        cp_own.start()

        x_rdmas = []
        for c in range(N_CHUNKS):
            row0 = (1 - my_y) * m + my_x * half_m + c * rows_c
            recv_view = pltpu.make_async_remote_copy(
                src_ref=send_buf.at[pl.ds(c * rows_c, rows_c), :],
                dst_ref=out_hbm.at[pl.ds(row0, rows_c), :],
                send_sem=ysend.at[c],
                recv_sem=yrecv.at[c],
                device_id=y_nbr,
                device_id_type=pl.DeviceIdType.MESH,
            )
            recv_view.wait_recv()
            fwd = pltpu.make_async_remote_copy(
                src_ref=out_hbm.at[pl.ds(row0, rows_c), :],
                dst_ref=out_hbm.at[pl.ds(row0, rows_c), :],
                send_sem=xsend.at[c],
                recv_sem=xrecv.at[c],
                device_id=x_nbr,
                device_id_type=pl.DeviceIdType.MESH,
            )
            fwd.start()
            x_rdmas.append(fwd)

        cp_own.wait()
        own_bf16[...] = own_f32[...].astype(jnp.bfloat16)
        cp_out = pltpu.make_async_copy(
            own_bf16,
            out_hbm.at[pl.ds(my_y * m, m), :],
            cp_sems.at[1],
        )
        cp_out.start()

        for c in range(N_CHUNKS):
            x_rdmas[c].wait_recv()
        for c in range(N_CHUNKS):
            y_rdmas[c].wait_send()
            x_rdmas[c].wait_send()
        cp_out.wait()

    return pl.pallas_call(
        body,
        out_shape=jax.ShapeDtypeStruct((N_Y * m, half_n), jnp.bfloat16),
        in_specs=[pl.BlockSpec(memory_space=pl.ANY)],
        out_specs=pl.BlockSpec(memory_space=pl.ANY),
        scratch_shapes=[
            pltpu.VMEM((half_m, half_n), jnp.float32),
            pltpu.VMEM((m, half_n), jnp.float32),
            pltpu.VMEM((m, half_n), jnp.bfloat16),
            pltpu.VMEM((half_m, half_n), jnp.bfloat16),
            pltpu.SemaphoreType.DMA((2,)),
            pltpu.SemaphoreType.DMA((N_CHUNKS,)),
            pltpu.SemaphoreType.DMA((N_CHUNKS,)),
            pltpu.SemaphoreType.DMA((N_CHUNKS,)),
            pltpu.SemaphoreType.DMA((N_CHUNKS,)),
            pltpu.SemaphoreType.DMA((N_CHUNKS,)),
        ],
        compiler_params=pltpu.CompilerParams(
            collective_id=0,
            vmem_limit_bytes=96 * 1024 * 1024,
        ),
    )(x)
